# baseline (device time: 2167524 ns/iter reference)
import jax
import jax.numpy as jnp
from jax import lax
from jax.experimental import pallas as pl
from jax.experimental.pallas import tpu as pltpu

M, N = 32768, 1024
H = M // 2
CH = 256
K = H // CH

_ANY = pl.ANY


def kernel(x):
    def body(x_ref, out_ref, recv_y, recv_x,
             y_send_sems, y_recv_sems, x_send_sems, x_recv_sems,
             a_vmem, b_vmem, o_vmem, a_sems, b_sems, st_sems, xcp_sems):
        my_x = lax.axis_index("x")
        my_y = lax.axis_index("y")
        y_nbr = (my_x, 1 - my_y)
        x_nbr = (1 - my_x, my_y)
        half_off = my_x * H
        other_off = (1 - my_x) * H

        barrier = pltpu.get_barrier_semaphore()
        for nbr in (y_nbr, x_nbr):
            pl.semaphore_signal(barrier, inc=1, device_id=nbr,
                                device_id_type=pl.DeviceIdType.MESH)
        pl.semaphore_wait(barrier, 2)

        y_rdmas = []
        for c in range(K):
            r = pltpu.make_async_remote_copy(
                src_ref=x_ref.at[pl.ds(half_off + c * CH, CH)],
                dst_ref=recv_y.at[pl.ds(c * CH, CH)],
                send_sem=y_send_sems.at[c],
                recv_sem=y_recv_sems.at[c],
                device_id=y_nbr,
                device_id_type=pl.DeviceIdType.MESH,
            )
            r.start()
            y_rdmas.append(r)

        x_rdmas = [None] * K
        st_cps = [None] * K
        xcp_cps = [None] * K

        def issue_xcp(idx):
            if idx >= 2:
                xcp_cps[idx - 2].wait()
            x_rdmas[idx].wait_recv()
            cp = pltpu.make_async_copy(
                recv_x.at[pl.ds(idx * CH, CH)],
                out_ref.at[pl.ds(other_off + idx * CH, CH)],
                xcp_sems.at[idx % 2],
            )
            cp.start()
            xcp_cps[idx] = cp

        for c in range(K):
            slot = c % 2
            sl = pl.ds(half_off + c * CH, CH)
            if c >= 2:
                st_cps[c - 2].wait()
                x_rdmas[c - 2].wait_send()
            cp_a = pltpu.make_async_copy(x_ref.at[sl], a_vmem.at[slot],
                                         a_sems.at[slot])
            cp_a.start()
            y_rdmas[c].wait_recv()
            cp_b = pltpu.make_async_copy(recv_y.at[pl.ds(c * CH, CH)],
                                         b_vmem.at[slot], b_sems.at[slot])
            cp_b.start()
            cp_a.wait()
            cp_b.wait()
            o_vmem[slot] = a_vmem[slot] + b_vmem[slot]
            st = pltpu.make_async_copy(o_vmem.at[slot], out_ref.at[sl],
                                       st_sems.at[slot])
            st.start()
            st_cps[c] = st
            rx = pltpu.make_async_remote_copy(
                src_ref=o_vmem.at[slot],
                dst_ref=recv_x.at[pl.ds(c * CH, CH)],
                send_sem=x_send_sems.at[c],
                recv_sem=x_recv_sems.at[c],
                device_id=x_nbr,
                device_id_type=pl.DeviceIdType.MESH,
            )
            rx.start()
            x_rdmas[c] = rx
            y_rdmas[c].wait_send()
            if c >= 2:
                issue_xcp(c - 2)

        for c in range(K - 2, K):
            st_cps[c].wait()
            x_rdmas[c].wait_send()
        issue_xcp(K - 2)
        issue_xcp(K - 1)
        xcp_cps[K - 2].wait()
        xcp_cps[K - 1].wait()

    out, _, _ = pl.pallas_call(
        body,
        out_shape=[
            jax.ShapeDtypeStruct((M, N), jnp.float32),
            jax.ShapeDtypeStruct((H, N), jnp.float32),
            jax.ShapeDtypeStruct((H, N), jnp.float32),
        ],
        in_specs=[pl.BlockSpec(memory_space=_ANY)],
        out_specs=[
            pl.BlockSpec(memory_space=_ANY),
            pl.BlockSpec(memory_space=_ANY),
            pl.BlockSpec(memory_space=_ANY),
        ],
        scratch_shapes=[
            pltpu.SemaphoreType.DMA((K,)),
            pltpu.SemaphoreType.DMA((K,)),
            pltpu.SemaphoreType.DMA((K,)),
            pltpu.SemaphoreType.DMA((K,)),
            pltpu.VMEM((2, CH, N), jnp.float32),
            pltpu.VMEM((2, CH, N), jnp.float32),
            pltpu.VMEM((2, CH, N), jnp.float32),
            pltpu.SemaphoreType.DMA((2,)),
            pltpu.SemaphoreType.DMA((2,)),
            pltpu.SemaphoreType.DMA((2,)),
            pltpu.SemaphoreType.DMA((2,)),
        ],
        compiler_params=pltpu.CompilerParams(collective_id=0),
    )(x)
    return out


# device time: 826992 ns/iter; 2.6210x vs baseline; 2.6210x over previous
import jax
import jax.numpy as jnp
from jax import lax
from jax.experimental import pallas as pl
from jax.experimental.pallas import tpu as pltpu

M, N = 32768, 1024
H = M // 2
CH = 256
K = H // CH

_ANY = pl.ANY


def kernel(x):
    def body(x_ref, out_ref,
             y_send_sems, y_recv_sems, x_send_sems, x_recv_sems,
             a_vmem, b_vmem, o_vmem, a_sems, b_sems, st_sems):
        my_x = lax.axis_index("x")
        my_y = lax.axis_index("y")
        y_nbr = (my_x, 1 - my_y)
        x_nbr = (1 - my_x, my_y)
        half_off = my_x * H

        barrier = pltpu.get_barrier_semaphore()
        for nbr in (y_nbr, x_nbr):
            pl.semaphore_signal(barrier, inc=1, device_id=nbr,
                                device_id_type=pl.DeviceIdType.MESH)
        pl.semaphore_wait(barrier, 2)

        y_rdmas = []
        for c in range(K):
            sl = pl.ds(half_off + c * CH, CH)
            r = pltpu.make_async_remote_copy(
                src_ref=x_ref.at[sl],
                dst_ref=out_ref.at[sl],
                send_sem=y_send_sems.at[c],
                recv_sem=y_recv_sems.at[c],
                device_id=y_nbr,
                device_id_type=pl.DeviceIdType.MESH,
            )
            r.start()
            y_rdmas.append(r)

        x_rdmas = [None] * K
        st_cps = [None] * K
        for c in range(K):
            slot = c % 2
            sl = pl.ds(half_off + c * CH, CH)
            if c >= 2:
                st_cps[c - 2].wait()
                x_rdmas[c - 2].wait_send()
            cp_a = pltpu.make_async_copy(x_ref.at[sl], a_vmem.at[slot],
                                         a_sems.at[slot])
            cp_a.start()
            y_rdmas[c].wait_recv()
            cp_b = pltpu.make_async_copy(out_ref.at[sl], b_vmem.at[slot],
                                         b_sems.at[slot])
            cp_b.start()
            cp_a.wait()
            cp_b.wait()
            o_vmem[slot] = a_vmem[slot] + b_vmem[slot]
            st = pltpu.make_async_copy(o_vmem.at[slot], out_ref.at[sl],
                                       st_sems.at[slot])
            st.start()
            st_cps[c] = st
            rx = pltpu.make_async_remote_copy(
                src_ref=o_vmem.at[slot],
                dst_ref=out_ref.at[sl],
                send_sem=x_send_sems.at[c],
                recv_sem=x_recv_sems.at[c],
                device_id=x_nbr,
                device_id_type=pl.DeviceIdType.MESH,
            )
            rx.start()
            x_rdmas[c] = rx
            y_rdmas[c].wait_send()

        for c in range(K - 2, K):
            st_cps[c].wait()
            x_rdmas[c].wait_send()
        for c in range(K):
            x_rdmas[c].wait_recv()

    return pl.pallas_call(
        body,
        out_shape=jax.ShapeDtypeStruct((M, N), jnp.float32),
        in_specs=[pl.BlockSpec(memory_space=_ANY)],
        out_specs=pl.BlockSpec(memory_space=_ANY),
        scratch_shapes=[
            pltpu.SemaphoreType.DMA((K,)),
            pltpu.SemaphoreType.DMA((K,)),
            pltpu.SemaphoreType.DMA((K,)),
            pltpu.SemaphoreType.DMA((K,)),
            pltpu.VMEM((2, CH, N), jnp.float32),
            pltpu.VMEM((2, CH, N), jnp.float32),
            pltpu.VMEM((2, CH, N), jnp.float32),
            pltpu.SemaphoreType.DMA((2,)),
            pltpu.SemaphoreType.DMA((2,)),
            pltpu.SemaphoreType.DMA((2,)),
        ],
        compiler_params=pltpu.CompilerParams(collective_id=0),
    )(x)
